# baseline (speedup 1.0000x reference)
import functools

import jax
import jax.numpy as jnp
from jax import lax
from jax.experimental import pallas as pl
from jax.experimental.pallas import tpu as pltpu
from jax.experimental.pallas import tpu_sc as plsc


def _sc_gather(table, idx):
    V, D = table.shape
    (B,) = idx.shape
    info = plsc.get_sparse_core_info()
    nc = info.num_cores
    ns = info.num_subcores
    nw = nc * ns
    b_per_w = B // nw
    half = b_per_w // 2
    lanes = info.num_lanes
    groups = half // lanes
    mesh = plsc.VectorSubcoreMesh(core_axis_name="c", subcore_axis_name="s")

    @functools.partial(
        pl.kernel,
        mesh=mesh,
        out_type=jax.ShapeDtypeStruct((B, D), jnp.float32),
        scratch_types=[
            pltpu.VMEM((b_per_w,), jnp.int32),
            pltpu.VMEM((half, D), jnp.float32),
            pltpu.VMEM_SHARED((ns, half, D), jnp.float32),
            pltpu.SemaphoreType.DMA,
            pltpu.SemaphoreType.DMA,
            pltpu.SemaphoreType.DMA,
        ],
    )
    def k(table_hbm, idx_hbm, out_hbm, idx_v, rows_v, shared, sem_i, sem_s, sem_d):
        s = lax.axis_index("s")
        wid = s * nc + lax.axis_index("c")
        base = wid * b_per_w
        pltpu.async_copy(idx_hbm.at[pl.ds(base, b_per_w)], idx_v, sem_i).wait()

        def group_body(g, _):
            vec_a = idx_v[pl.ds(g * lanes, lanes)]
            vec_b = idx_v[pl.ds(half + g * lanes, lanes)]
            for l in range(lanes):
                pltpu.async_copy(
                    table_hbm.at[pl.ds(vec_a[l], 1), :],
                    rows_v.at[pl.ds(g * lanes + l, 1), :],
                    sem_s,
                )
                pltpu.async_copy(
                    table_hbm.at[pl.ds(vec_b[l], 1), :],
                    shared.at[s, pl.ds(g * lanes + l, 1), :],
                    sem_d,
                )
            return 0

        lax.fori_loop(0, groups, group_body, 0)
        pltpu.make_async_copy(
            table_hbm.at[pl.ds(0, half), :], rows_v, sem_s
        ).wait()
        pltpu.make_async_copy(
            table_hbm.at[pl.ds(0, half), :], shared.at[s], sem_d
        ).wait()
        pltpu.sync_copy(rows_v, out_hbm.at[pl.ds(base, half)])
        pltpu.sync_copy(shared.at[s], out_hbm.at[pl.ds(base + half, half)])

    return k(table, idx)


def _tc_project(x, W, b):
    B, D = x.shape
    H = W.shape[1]
    blk = 2048

    def body(x_ref, w_ref, b_ref, o_ref):
        o_ref[...] = (
            jnp.dot(x_ref[...], w_ref[...], preferred_element_type=jnp.float32)
            + b_ref[...]
        )

    return pl.pallas_call(
        body,
        grid=(B // blk,),
        in_specs=[
            pl.BlockSpec((blk, D), lambda i: (i, 0)),
            pl.BlockSpec((D, H), lambda i: (0, 0)),
            pl.BlockSpec((1, H), lambda i: (0, 0)),
        ],
        out_specs=pl.BlockSpec((blk, H), lambda i: (i, 0)),
        out_shape=jax.ShapeDtypeStruct((B, H), jnp.float32),
    )(x, W, b.reshape(1, H))


def kernel(task_ids, table, W, b):
    rows = _sc_gather(table, task_ids.astype(jnp.int32))
    return _tc_project(rows, W, b)


# speedup vs baseline: 1.1991x; 1.1991x over previous
import functools

import jax
import jax.numpy as jnp
from jax import lax
from jax.experimental import pallas as pl
from jax.experimental.pallas import tpu as pltpu
from jax.experimental.pallas import tpu_sc as plsc


def _sc_gather(table, idx):
    V, D = table.shape
    (B,) = idx.shape
    info = plsc.get_sparse_core_info()
    nc = info.num_cores
    nw = nc * info.num_subcores
    b_per_w = B // nw
    lanes = info.num_lanes
    groups = b_per_w // lanes
    mesh = plsc.VectorSubcoreMesh(core_axis_name="c", subcore_axis_name="s")

    @functools.partial(
        pl.kernel,
        mesh=mesh,
        out_type=jax.ShapeDtypeStruct((B, D), jnp.float32),
        scratch_types=[
            pltpu.VMEM((b_per_w,), jnp.int32),
            pltpu.VMEM((b_per_w, D), jnp.float32),
            pltpu.SemaphoreType.DMA,
            pltpu.SemaphoreType.DMA,
            pltpu.SemaphoreType.DMA,
            pltpu.SemaphoreType.DMA,
            pltpu.SemaphoreType.DMA,
        ],
    )
    def k(table_hbm, idx_hbm, out_hbm, idx_v, rows_v, sem_i, s0, s1, s2, s3):
        sems = [s0, s1, s2, s3]
        wid = lax.axis_index("s") * nc + lax.axis_index("c")
        base = wid * b_per_w
        pltpu.async_copy(idx_hbm.at[pl.ds(base, b_per_w)], idx_v, sem_i).wait()

        def group_body(g, _):
            vec = idx_v[pl.ds(g * lanes, lanes)]
            for l in range(lanes):
                r = vec[l]
                pltpu.async_copy(
                    table_hbm.at[pl.ds(r, 1), :],
                    rows_v.at[pl.ds(g * lanes + l, 1), :],
                    sems[l % 4],
                )
            return 0

        lax.fori_loop(0, groups, group_body, 0)
        for q in range(4):
            pltpu.make_async_copy(
                table_hbm.at[pl.ds(0, b_per_w // 4), :],
                rows_v.at[pl.ds(0, b_per_w // 4), :],
                sems[q],
            ).wait()
        pltpu.sync_copy(rows_v, out_hbm.at[pl.ds(base, b_per_w)])

    return k(table, idx)


def _tc_project(x, W, b):
    B, D = x.shape
    H = W.shape[1]
    blk = 2048

    def body(x_ref, w_ref, b_ref, o_ref):
        o_ref[...] = (
            jnp.dot(x_ref[...], w_ref[...], preferred_element_type=jnp.float32)
            + b_ref[...]
        )

    return pl.pallas_call(
        body,
        grid=(B // blk,),
        in_specs=[
            pl.BlockSpec((blk, D), lambda i: (i, 0)),
            pl.BlockSpec((D, H), lambda i: (0, 0)),
            pl.BlockSpec((1, H), lambda i: (0, 0)),
        ],
        out_specs=pl.BlockSpec((blk, H), lambda i: (i, 0)),
        out_shape=jax.ShapeDtypeStruct((B, H), jnp.float32),
    )(x, W, b.reshape(1, H))


def kernel(task_ids, table, W, b):
    rows = _sc_gather(table, task_ids.astype(jnp.int32))
    return _tc_project(rows, W, b)
